# baseline (speedup 1.0000x reference)
import jax
import jax.numpy as jnp
from jax import lax
from jax.experimental import pallas as pl
from jax.experimental.pallas import tpu as pltpu
from jax.experimental.pallas import tpu_sc as plsc

N = 100000
D = 128
LANES = 16
L = 8
NC = 2
NS = 16
NW = NC * NS
C = 8
NBUF = 3
NCHUNK = N // C
JLO = NCHUNK // NW
NHI = NCHUNK - JLO * NW
JHI = JLO + 1


def _body(tok_hbm, nid_hbm, ttab_hbm, ntab_hbm, out_hbm,
          tok_idx_v, nid_v, rows_v, nrows_v, out_v, sem_t, sem_n, sem_o):
    wid = lax.axis_index("s") * NC + lax.axis_index("c")
    my = JLO + jnp.where(wid < NHI, 1, 0)
    base_chunk = wid * JLO + jnp.minimum(wid, NHI)

    @pl.when(wid < NHI)
    def _():
        pltpu.sync_copy(
            tok_hbm.at[pl.ds(base_chunk * C * L, JHI * C * L)],
            tok_idx_v.at[pl.ds(0, JHI * C * L)])
        pltpu.sync_copy(
            nid_hbm.at[pl.ds(base_chunk * C, JHI * C)],
            nid_v.at[pl.ds(0, JHI * C)])

    @pl.when(wid >= NHI)
    def _():
        pltpu.sync_copy(
            tok_hbm.at[pl.ds(base_chunk * C * L, JLO * C * L)],
            tok_idx_v.at[pl.ds(0, JLO * C * L)])
        pltpu.sync_copy(
            nid_hbm.at[pl.ds(base_chunk * C, JLO * C)],
            nid_v.at[pl.ds(0, JLO * C)])

    def start(j, b):
        pltpu.async_copy(
            ttab_hbm.at[tok_idx_v.at[pl.ds(j * C * L, C * L)]],
            rows_v.at[b], sem_t.at[b])
        pltpu.async_copy(
            ntab_hbm.at[nid_v.at[pl.ds(j * C, C)]],
            nrows_v.at[b], sem_n.at[b])

    def drain(j, b):
        pltpu.make_async_copy(
            ttab_hbm.at[tok_idx_v.at[pl.ds(j * C * L, C * L)]],
            rows_v.at[b], sem_t.at[b]).wait()
        pltpu.make_async_copy(
            ntab_hbm.at[nid_v.at[pl.ds(j * C, C)]],
            nrows_v.at[b], sem_n.at[b]).wait()

    def drain_out(j, b):
        pltpu.make_async_copy(
            out_v.at[b], out_hbm.at[pl.ds((base_chunk + j) * C, C)],
            sem_o.at[b]).wait()

    def compute(j, b):
        @pl.when(j >= NBUF)
        def _():
            drain_out(j - NBUF, b)

        @plsc.parallel_loop(0, C, step=1, unroll=4)
        def row(r):
            base = r * L
            for h in range(D // LANES):
                sl = pl.ds(h * LANES, LANES)
                t0 = rows_v[b, base + 0, sl] + rows_v[b, base + 1, sl]
                t1 = rows_v[b, base + 2, sl] + rows_v[b, base + 3, sl]
                t2 = rows_v[b, base + 4, sl] + rows_v[b, base + 5, sl]
                t3 = rows_v[b, base + 6, sl] + rows_v[b, base + 7, sl]
                out_v[b, r, sl] = (t0 + t1) + ((t2 + t3) + nrows_v[b, r, sl])

        pltpu.async_copy(
            out_v.at[b], out_hbm.at[pl.ds((base_chunk + j) * C, C)],
            sem_o.at[b])

    for b in range(NBUF - 1):
        start(b, b)

    def step(j, carry):
        b = lax.rem(j, NBUF)

        @pl.when(j + NBUF - 1 < my)
        def _():
            start(j + NBUF - 1, lax.rem(j + NBUF - 1, NBUF))

        drain(j, b)
        compute(j, b)
        return carry

    lax.fori_loop(0, my, step, 0)

    def tail(k, carry):
        j = my - NBUF + k
        drain_out(j, lax.rem(j, NBUF))
        return carry

    lax.fori_loop(0, NBUF, tail, 0)


@jax.jit
def _embed(tokens_flat, nids, ttab, ntab):
    mesh = plsc.VectorSubcoreMesh(core_axis_name="c", subcore_axis_name="s")
    f = pl.kernel(
        _body,
        out_type=jax.ShapeDtypeStruct((N, D), jnp.float32),
        mesh=mesh,
        compiler_params=pltpu.CompilerParams(use_tc_tiling_on_sc=False),
        scratch_types=[
            pltpu.VMEM((JHI * C * L,), jnp.int32),
            pltpu.VMEM((JHI * C,), jnp.int32),
            pltpu.VMEM((NBUF, C * L, D), jnp.float32),
            pltpu.VMEM((NBUF, C, D), jnp.float32),
            pltpu.VMEM((NBUF, C, D), jnp.float32),
            pltpu.SemaphoreType.DMA((NBUF,)),
            pltpu.SemaphoreType.DMA((NBUF,)),
            pltpu.SemaphoreType.DMA((NBUF,)),
        ],
    )
    return f(tokens_flat, nids, ttab, ntab)


def kernel(tokens, node_ids, token_table, node_table):
    return _embed(tokens.reshape(-1), node_ids, token_table, node_table)


# speedup vs baseline: 1.1277x; 1.1277x over previous
import jax
import jax.numpy as jnp
from jax import lax
from jax.experimental import pallas as pl
from jax.experimental.pallas import tpu as pltpu
from jax.experimental.pallas import tpu_sc as plsc

N = 100000
D = 128
LANES = 16
L = 8
NC = 2
NS = 16
NW = NC * NS
C = 16
NBUF = 3
NCHUNK = N // C
JLO = NCHUNK // NW
NHI = NCHUNK - JLO * NW
JHI = JLO + 1


def _body(tok_hbm, nid_hbm, ttab_hbm, ntab_hbm, out_hbm,
          tok_idx_v, nid_v, rows_v, nrows_v, out_v, sem_t, sem_n, sem_o):
    wid = lax.axis_index("s") * NC + lax.axis_index("c")
    my = JLO + jnp.where(wid < NHI, 1, 0)
    base_chunk = wid * JLO + jnp.minimum(wid, NHI)

    @pl.when(wid < NHI)
    def _():
        pltpu.sync_copy(
            tok_hbm.at[pl.ds(base_chunk * C * L, JHI * C * L)],
            tok_idx_v.at[pl.ds(0, JHI * C * L)])
        pltpu.sync_copy(
            nid_hbm.at[pl.ds(base_chunk * C, JHI * C)],
            nid_v.at[pl.ds(0, JHI * C)])

    @pl.when(wid >= NHI)
    def _():
        pltpu.sync_copy(
            tok_hbm.at[pl.ds(base_chunk * C * L, JLO * C * L)],
            tok_idx_v.at[pl.ds(0, JLO * C * L)])
        pltpu.sync_copy(
            nid_hbm.at[pl.ds(base_chunk * C, JLO * C)],
            nid_v.at[pl.ds(0, JLO * C)])

    def start(j, b):
        pltpu.async_copy(
            ttab_hbm.at[tok_idx_v.at[pl.ds(j * C * L, C * L)]],
            rows_v.at[b], sem_t.at[b])
        pltpu.async_copy(
            ntab_hbm.at[nid_v.at[pl.ds(j * C, C)]],
            nrows_v.at[b], sem_n.at[b])

    def drain(j, b):
        pltpu.make_async_copy(
            ttab_hbm.at[tok_idx_v.at[pl.ds(j * C * L, C * L)]],
            rows_v.at[b], sem_t.at[b]).wait()
        pltpu.make_async_copy(
            ntab_hbm.at[nid_v.at[pl.ds(j * C, C)]],
            nrows_v.at[b], sem_n.at[b]).wait()

    def drain_out(j, b):
        pltpu.make_async_copy(
            out_v.at[b], out_hbm.at[pl.ds((base_chunk + j) * C, C)],
            sem_o.at[b]).wait()

    def compute(j, b):
        @pl.when(j >= NBUF)
        def _():
            drain_out(j - NBUF, b)

        @plsc.parallel_loop(0, C, step=1, unroll=2)
        def row(r):
            base = r * L
            for h in range(D // LANES):
                sl = pl.ds(h * LANES, LANES)
                t0 = rows_v[b, base + 0, sl] + rows_v[b, base + 1, sl]
                t1 = rows_v[b, base + 2, sl] + rows_v[b, base + 3, sl]
                t2 = rows_v[b, base + 4, sl] + rows_v[b, base + 5, sl]
                t3 = rows_v[b, base + 6, sl] + rows_v[b, base + 7, sl]
                out_v[b, r, sl] = (t0 + t1) + ((t2 + t3) + nrows_v[b, r, sl])

        pltpu.async_copy(
            out_v.at[b], out_hbm.at[pl.ds((base_chunk + j) * C, C)],
            sem_o.at[b])

    for b in range(NBUF - 1):
        start(b, b)

    def step(j, carry):
        b = lax.rem(j, NBUF)

        @pl.when(j + NBUF - 1 < my)
        def _():
            start(j + NBUF - 1, lax.rem(j + NBUF - 1, NBUF))

        drain(j, b)
        compute(j, b)
        return carry

    lax.fori_loop(0, my, step, 0)

    def tail(k, carry):
        j = my - NBUF + k
        drain_out(j, lax.rem(j, NBUF))
        return carry

    lax.fori_loop(0, NBUF, tail, 0)


@jax.jit
def _embed(tokens_flat, nids, ttab, ntab):
    mesh = plsc.VectorSubcoreMesh(core_axis_name="c", subcore_axis_name="s")
    f = pl.kernel(
        _body,
        out_type=jax.ShapeDtypeStruct((N, D), jnp.float32),
        mesh=mesh,
        compiler_params=pltpu.CompilerParams(use_tc_tiling_on_sc=False),
        scratch_types=[
            pltpu.VMEM((JHI * C * L,), jnp.int32),
            pltpu.VMEM((JHI * C,), jnp.int32),
            pltpu.VMEM((NBUF, C * L, D), jnp.float32),
            pltpu.VMEM((NBUF, C, D), jnp.float32),
            pltpu.VMEM((NBUF, C, D), jnp.float32),
            pltpu.SemaphoreType.DMA((NBUF,)),
            pltpu.SemaphoreType.DMA((NBUF,)),
            pltpu.SemaphoreType.DMA((NBUF,)),
        ],
    )
    return f(tokens_flat, nids, ttab, ntab)


def kernel(tokens, node_ids, token_table, node_table):
    return _embed(tokens.reshape(-1), node_ids, token_table, node_table)
